# baseline (speedup 1.0000x reference)
import functools

import jax
import jax.numpy as jnp
from jax import lax
from jax.experimental import pallas as pl
from jax.experimental.pallas import tpu as pltpu
from jax.experimental.pallas import tpu_sc as plsc

N_NODES = 10000
N_EDGES = 320000
D = 128
DH = D // 2

NC = 2
NS = 16
NW = NC * NS
NP = 10016
RPT = NP // NS
DUMP = 10008
CHUNK = 512

E64 = 20480
NCH64 = E64 // CHUNK
NSEC64 = 4

E16 = 10240
NCH16 = E16 // CHUNK
NSEC16 = 1

_SC_PARAMS = pltpu.CompilerParams(use_tc_tiling_on_sc=False)


def _make_pass(width, nch, nsec, feature_split):
    sch = nch // nsec
    ept = nch * CHUNK
    sept = sch * CHUNK
    nt = sch // 2
    mesh = plsc.VectorSubcoreMesh(core_axis_name="c", subcore_axis_name="s")

    @functools.partial(
        pl.kernel,
        out_type=jax.ShapeDtypeStruct((NC, NP, width), jnp.float32),
        mesh=mesh,
        scratch_types=[
            pltpu.VMEM((sept,), jnp.int32),
            pltpu.VMEM((sept,), jnp.int32),
            pltpu.VMEM((CHUNK, width), jnp.float32),
            pltpu.VMEM((CHUNK, width), jnp.float32),
            pltpu.VMEM_SHARED((NP, width), jnp.float32),
            pltpu.SemaphoreType.DMA,
            pltpu.SemaphoreType.DMA,
            pltpu.SemaphoreType.DMA,
            pltpu.SemaphoreType.DMA,
        ],
        compiler_params=_SC_PARAMS,
    )
    def seg(table_hbm, srclo_hbm, srchi_hbm, dst_hbm, zeros_hbm, out_hbm,
            src_v, dst_v, buf0, buf1, acc_sh, gsem0, gsem1, ssem0, ssem1):
        c = lax.axis_index("c")
        s = lax.axis_index("s")

        def idx(ref, chunk_no):
            return ref.at[pl.ds(chunk_no * CHUNK, CHUNK)]

        pltpu.sync_copy(zeros_hbm, buf1)
        nfull = RPT // CHUNK
        for j in range(nfull):
            pltpu.sync_copy(buf1, acc_sh.at[pl.ds(s * RPT + j * CHUNK, CHUNK)])
        rem = RPT - nfull * CHUNK
        if rem:
            pltpu.sync_copy(buf1.at[pl.ds(0, rem)],
                            acc_sh.at[pl.ds(s * RPT + nfull * CHUNK, rem)])
        plsc.subcore_barrier()

        def section(sec, carry):
            if feature_split:
                base = s * ept + sec * sept

                @pl.when(c == 0)
                def _():
                    pltpu.sync_copy(srclo_hbm.at[pl.ds(base, sept)], src_v)

                @pl.when(c == 1)
                def _():
                    pltpu.sync_copy(srchi_hbm.at[pl.ds(base, sept)], src_v)
            else:
                base = (s * NC + c) * ept + sec * sept
                pltpu.sync_copy(srclo_hbm.at[pl.ds(base, sept)], src_v)
            pltpu.sync_copy(dst_hbm.at[pl.ds(base, sept)], dst_v)
            pltpu.async_copy(table_hbm.at[idx(src_v, 0)], buf0, gsem0)

            def body(t, carry2):
                g0 = 2 * t
                g1 = g0 + 1
                pltpu.async_copy(table_hbm.at[idx(src_v, g1)], buf1, gsem1)
                pltpu.make_async_copy(table_hbm.at[idx(src_v, 0)],
                                      buf0, gsem0).wait()
                pltpu.async_copy(buf0, acc_sh.at[idx(dst_v, g0)],
                                 ssem0, add=True)
                pltpu.make_async_copy(table_hbm.at[idx(src_v, 0)],
                                      buf1, gsem1).wait()
                pltpu.async_copy(buf1, acc_sh.at[idx(dst_v, g1)],
                                 ssem1, add=True)
                pltpu.make_async_copy(buf0, acc_sh.at[idx(dst_v, 0)],
                                      ssem0).wait()

                @pl.when(t + 1 < nt)
                def _():
                    pltpu.async_copy(table_hbm.at[idx(src_v, g0 + 2)],
                                     buf0, gsem0)

                pltpu.make_async_copy(buf1, acc_sh.at[idx(dst_v, 0)],
                                      ssem1).wait()
                return carry2

            lax.fori_loop(0, nt, body, 0)
            return carry

        lax.fori_loop(0, nsec, section, 0)
        plsc.subcore_barrier()
        pltpu.sync_copy(acc_sh.at[pl.ds(s * RPT, RPT)],
                        out_hbm.at[c, pl.ds(s * RPT, RPT)])

    return seg


_seg64 = _make_pass(DH, NCH64, NSEC64, True)
_seg16 = _make_pass(16, NCH16, NSEC16, False)


def _deg_inv(deg_ref):
    deg = deg_ref[0, :, 0:1] + deg_ref[1, :, 0:1]
    return 1.0 / jnp.maximum(deg, 1.0)


def _layer1_body(acc_ref, deg_ref, x_ref, wl_ref, wr_ref, b_ref, o_ref):
    inv = _deg_inv(deg_ref)
    agg = jnp.concatenate([acc_ref[0], acc_ref[1]], axis=1) * inv
    h = (jnp.dot(agg, wl_ref[...], preferred_element_type=jnp.float32)
         + jnp.dot(x_ref[...], wr_ref[...], preferred_element_type=jnp.float32)
         + b_ref[...])
    h = jnp.maximum(h, 0.0)
    o_ref[0] = h[:, :DH]
    o_ref[1] = h[:, DH:]


def _layer1(acc, deg, x, wl, wr, b):
    return pl.pallas_call(
        _layer1_body,
        out_shape=jax.ShapeDtypeStruct((2, NP, DH), jnp.float32),
    )(acc, deg, x, wl, wr, b)


def _layer2_body(acc_ref, deg_ref, h_ref, wl_ref, wr_ref, b_ref,
                 wl3_ref, wr3_ref, b3_ref, p_ref, q_ref):
    inv = _deg_inv(deg_ref)
    agg = jnp.concatenate([acc_ref[0], acc_ref[1]], axis=1) * inv
    h1 = jnp.concatenate([h_ref[0], h_ref[1]], axis=1)
    h = (jnp.dot(agg, wl_ref[...], preferred_element_type=jnp.float32)
         + jnp.dot(h1, wr_ref[...], preferred_element_type=jnp.float32)
         + b_ref[...])
    h = jnp.maximum(h, 0.0)
    p_ref[...] = jnp.dot(h, wl3_ref[...], preferred_element_type=jnp.float32)
    q_ref[...] = (jnp.dot(h, wr3_ref[...], preferred_element_type=jnp.float32)
                  + b3_ref[...])


def _layer2(acc, deg, h, wl, wr, b, wl3, wr3, b3):
    return pl.pallas_call(
        _layer2_body,
        out_shape=[jax.ShapeDtypeStruct((NP, 16), jnp.float32),
                   jax.ShapeDtypeStruct((NP, 16), jnp.float32)],
    )(acc, deg, h, wl, wr, b, wl3, wr3, b3)


def _final_body(acc_ref, deg_ref, q_ref, o_ref):
    inv = _deg_inv(deg_ref)
    z = (acc_ref[0] + acc_ref[1]) * inv + q_ref[...]
    z0 = z[:, 0:1]
    z1 = z[:, 1:2]
    m = jnp.maximum(z0, z1)
    lse = m + jnp.log(jnp.exp(z0 - m) + jnp.exp(z1 - m))
    o_ref[...] = jnp.concatenate([z0 - lse, z1 - lse], axis=1)


def _final(acc, deg, q):
    return pl.pallas_call(
        _final_body,
        out_shape=jax.ShapeDtypeStruct((NP, 2), jnp.float32),
    )(acc, deg, q)


def _pad_edges(a, per_tile, per_tile_pad, ntiles, fill):
    a = a.reshape(ntiles, per_tile)
    return jnp.pad(a, ((0, 0), (0, per_tile_pad - per_tile)),
                   constant_values=fill).reshape(-1)


def kernel(x, edge_index, Wl1, Wr1, b1, Wl2, Wr2, b2, Wl3, Wr3, b3):
    ei = edge_index.astype(jnp.int32)
    src = ei[0]
    dst = ei[1]

    ept64 = N_EDGES // NS
    src64 = _pad_edges(src, ept64, E64, NS, 0)
    src64_hi = src64 + NP
    dst64 = _pad_edges(dst, ept64, E64, NS, DUMP)
    ept16 = N_EDGES // NW
    src16 = _pad_edges(src, ept16, E16, NW, 0)
    dst16 = _pad_edges(dst, ept16, E16, NW, DUMP)

    x_p = jnp.pad(x.astype(jnp.float32), ((0, NP - N_NODES), (0, 0)))
    x_stack = jnp.concatenate([x_p[:, :DH], x_p[:, DH:]], axis=0)
    zeros64 = jnp.zeros((CHUNK, DH), jnp.float32)
    zeros16 = jnp.zeros((CHUNK, 16), jnp.float32)

    e0_table = jnp.zeros((NP, 16), jnp.float32).at[:, 0].set(1.0)
    deg = _seg16(e0_table, src16, src16, dst16, zeros16)

    s1 = _seg64(x_stack, src64, src64_hi, dst64, zeros64)
    h1 = _layer1(s1, deg, x_p, Wl1, Wr1, b1.reshape(1, D))

    wl3_16 = jnp.pad(Wl3, ((0, 0), (0, 14)))
    wr3_16 = jnp.pad(Wr3, ((0, 0), (0, 14)))
    b3_16 = jnp.pad(b3, (0, 14)).reshape(1, 16)
    s2 = _seg64(h1.reshape(2 * NP, DH), src64, src64_hi, dst64, zeros64)
    p16, q16 = _layer2(s2, deg, h1, Wl2, Wr2, b2.reshape(1, D),
                       wl3_16, wr3_16, b3_16)

    s3 = _seg16(p16, src16, src16, dst16, zeros16)
    out = _final(s3, deg, q16)
    return out[:N_NODES]


# speedup vs baseline: 1.0846x; 1.0846x over previous
import functools

import jax
import jax.numpy as jnp
from jax import lax
from jax.experimental import pallas as pl
from jax.experimental.pallas import tpu as pltpu
from jax.experimental.pallas import tpu_sc as plsc

N_NODES = 10000
N_EDGES = 320000
D = 128
DH = D // 2

NC = 2
NS = 16
NW = NC * NS
NP = 10016
RPT = NP // NS
DUMP = 10008
CHUNK = 128
L = 16

E64 = 20480
K64 = 2
E16 = 10240
K16 = 8

_SC_PARAMS = pltpu.CompilerParams(use_tc_tiling_on_sc=False,
                                  needs_layout_passes=False)


def _make_pass(width, ept, k, feature_split, with_deg):
    nch = ept // CHUNK
    nt = nch // (2 * k)
    mesh = plsc.VectorSubcoreMesh(core_axis_name="c", subcore_axis_name="s")

    out_type = jax.ShapeDtypeStruct((NC, NP, width), jnp.float32)
    scratch = [
        pltpu.VMEM((ept,), jnp.int32),
        pltpu.VMEM((ept,), jnp.int32),
        pltpu.VMEM((k, CHUNK, width), jnp.float32),
        pltpu.VMEM((k, CHUNK, width), jnp.float32),
        pltpu.VMEM_SHARED((NP, width), jnp.float32),
        pltpu.SemaphoreType.DMA,
        pltpu.SemaphoreType.DMA,
        pltpu.SemaphoreType.DMA,
        pltpu.SemaphoreType.DMA,
    ]
    if with_deg:
        out_type = [out_type, jax.ShapeDtypeStruct((NS, NP), jnp.float32)]
        scratch.insert(4, pltpu.VMEM((NP,), jnp.float32))

    @functools.partial(pl.kernel, out_type=out_type, mesh=mesh,
                       scratch_types=scratch, compiler_params=_SC_PARAMS)
    def seg(*refs):
        if with_deg:
            (table_hbm, srclo_hbm, srchi_hbm, dst_hbm, zeros_hbm, znp_hbm,
             out_hbm, deg_hbm,
             src_v, dst_v, buf0, buf1, deg_v, acc_sh,
             gsem0, gsem1, ssem0, ssem1) = refs
        else:
            (table_hbm, srclo_hbm, srchi_hbm, dst_hbm, zeros_hbm,
             out_hbm,
             src_v, dst_v, buf0, buf1, acc_sh,
             gsem0, gsem1, ssem0, ssem1) = refs
        c = lax.axis_index("c")
        s = lax.axis_index("s")
        if feature_split:
            base = s * ept

            @pl.when(c == 0)
            def _():
                pltpu.sync_copy(srclo_hbm.at[pl.ds(base, ept)], src_v)

            @pl.when(c == 1)
            def _():
                pltpu.sync_copy(srchi_hbm.at[pl.ds(base, ept)], src_v)
        else:
            base = (s * NC + c) * ept
            pltpu.sync_copy(srclo_hbm.at[pl.ds(base, ept)], src_v)
        pltpu.sync_copy(dst_hbm.at[pl.ds(base, ept)], dst_v)
        if with_deg:
            pltpu.sync_copy(znp_hbm, deg_v)

        def idx(ref, chunk_no):
            return ref.at[pl.ds(chunk_no * CHUNK, CHUNK)]

        def gather_k(group, buf, sem):
            for j in range(k):
                pltpu.async_copy(table_hbm.at[idx(src_v, group * k + j)],
                                 buf.at[j], sem)

        def gwait_k(buf, sem):
            d = pltpu.make_async_copy(table_hbm.at[idx(src_v, 0)],
                                      buf.at[0], sem)
            for _ in range(k):
                d.wait()

        def scatter_k(group, buf, sem):
            for j in range(k):
                pltpu.async_copy(buf.at[j],
                                 acc_sh.at[idx(dst_v, group * k + j)],
                                 sem, add=True)

        def swait_k(buf, sem):
            d = pltpu.make_async_copy(buf.at[0], acc_sh.at[idx(dst_v, 0)], sem)
            for _ in range(k):
                d.wait()

        gather_k(0, buf0, gsem0)
        pltpu.sync_copy(zeros_hbm, buf1)
        nfull = RPT // CHUNK
        for j in range(nfull):
            pltpu.sync_copy(buf1.at[j % k],
                            acc_sh.at[pl.ds(s * RPT + j * CHUNK, CHUNK)])
        rem = RPT - nfull * CHUNK
        if rem:
            pltpu.sync_copy(buf1.at[0].at[pl.ds(0, rem)],
                            acc_sh.at[pl.ds(s * RPT + nfull * CHUNK, rem)])
        plsc.subcore_barrier()

        epg = 2 * k * CHUNK

        def body(t, carry):
            g0 = 2 * t
            g1 = g0 + 1
            gather_k(g1, buf1, gsem1)
            gwait_k(buf0, gsem0)
            scatter_k(g0, buf0, ssem0)
            gwait_k(buf1, gsem1)
            scatter_k(g1, buf1, ssem1)
            if with_deg:
                @pl.when(c == 0)
                def _():
                    ones = jnp.full((L,), 1.0, jnp.float32)

                    def dbody(i, carry2):
                        dd = dst_v[pl.ds(t * epg + i * L, L)]
                        plsc.addupdate_scatter(deg_v, [dd], ones)
                        return carry2

                    lax.fori_loop(0, epg // L, dbody, 0)
            swait_k(buf0, ssem0)

            @pl.when(t + 1 < nt)
            def _():
                gather_k(g0 + 2, buf0, gsem0)

            swait_k(buf1, ssem1)
            return carry

        lax.fori_loop(0, nt, body, 0)
        plsc.subcore_barrier()
        pltpu.sync_copy(acc_sh.at[pl.ds(s * RPT, RPT)],
                        out_hbm.at[c, pl.ds(s * RPT, RPT)])
        if with_deg:
            @pl.when(c == 0)
            def _():
                pltpu.sync_copy(deg_v, deg_hbm.at[s])

    return seg


_seg64_deg = _make_pass(DH, E64, K64, True, True)
_seg64 = _make_pass(DH, E64, K64, True, False)
_seg16 = _make_pass(16, E16, K16, False, False)


def _inv_deg(deg_ref):
    deg = jnp.sum(deg_ref[...], axis=0)[:, None]
    return 1.0 / jnp.maximum(deg, 1.0)


def _self_body(x_ref, wr_ref, b_ref, o_ref):
    o_ref[...] = (jnp.dot(x_ref[...], wr_ref[...],
                          preferred_element_type=jnp.float32) + b_ref[...])


def _self(x, wr, b):
    return pl.pallas_call(
        _self_body,
        out_shape=jax.ShapeDtypeStruct((NP, D), jnp.float32),
    )(x, wr, b)


def _layer1_body(acc_ref, deg_ref, xr_ref, wl_ref, o_ref):
    inv = _inv_deg(deg_ref)
    agg = jnp.concatenate([acc_ref[0], acc_ref[1]], axis=1) * inv
    h = (jnp.dot(agg, wl_ref[...], preferred_element_type=jnp.float32)
         + xr_ref[...])
    h = jnp.maximum(h, 0.0)
    o_ref[0] = h[:, :DH]
    o_ref[1] = h[:, DH:]


def _layer1(acc, deg, xr, wl):
    return pl.pallas_call(
        _layer1_body,
        out_shape=jax.ShapeDtypeStruct((2, NP, DH), jnp.float32),
    )(acc, deg, xr, wl)


def _self2_body(h_ref, wr_ref, b_ref, o_ref):
    h1 = jnp.concatenate([h_ref[0], h_ref[1]], axis=1)
    o_ref[...] = (jnp.dot(h1, wr_ref[...],
                          preferred_element_type=jnp.float32) + b_ref[...])


def _self2(h, wr, b):
    return pl.pallas_call(
        _self2_body,
        out_shape=jax.ShapeDtypeStruct((NP, D), jnp.float32),
    )(h, wr, b)


def _layer2_body(acc_ref, deg_ref, hr_ref, wl_ref,
                 wl3_ref, wr3_ref, b3_ref, p_ref, q_ref):
    inv = _inv_deg(deg_ref)
    agg = jnp.concatenate([acc_ref[0], acc_ref[1]], axis=1) * inv
    h = (jnp.dot(agg, wl_ref[...], preferred_element_type=jnp.float32)
         + hr_ref[...])
    h = jnp.maximum(h, 0.0)
    p_ref[...] = jnp.dot(h, wl3_ref[...], preferred_element_type=jnp.float32)
    q_ref[...] = (jnp.dot(h, wr3_ref[...], preferred_element_type=jnp.float32)
                  + b3_ref[...])


def _layer2(acc, deg, hr, wl, wl3, wr3, b3):
    return pl.pallas_call(
        _layer2_body,
        out_shape=[jax.ShapeDtypeStruct((NP, 16), jnp.float32),
                   jax.ShapeDtypeStruct((NP, 16), jnp.float32)],
    )(acc, deg, hr, wl, wl3, wr3, b3)


def _final_body(acc_ref, deg_ref, q_ref, o_ref):
    inv = _inv_deg(deg_ref)
    z = (acc_ref[0] + acc_ref[1]) * inv + q_ref[...]
    z0 = z[:, 0:1]
    z1 = z[:, 1:2]
    m = jnp.maximum(z0, z1)
    lse = m + jnp.log(jnp.exp(z0 - m) + jnp.exp(z1 - m))
    o_ref[...] = jnp.concatenate([z0 - lse, z1 - lse], axis=1)


def _final(acc, deg, q):
    return pl.pallas_call(
        _final_body,
        out_shape=jax.ShapeDtypeStruct((NP, 2), jnp.float32),
    )(acc, deg, q)


def _pad_edges(a, per_tile, per_tile_pad, ntiles, fill):
    a = a.reshape(ntiles, per_tile)
    return jnp.pad(a, ((0, 0), (0, per_tile_pad - per_tile)),
                   constant_values=fill).reshape(-1)


def kernel(x, edge_index, Wl1, Wr1, b1, Wl2, Wr2, b2, Wl3, Wr3, b3):
    ei = edge_index.astype(jnp.int32)
    src = ei[0]
    dst = ei[1]

    ept64 = N_EDGES // NS
    src64 = _pad_edges(src, ept64, E64, NS, 0)
    src64_hi = src64 + NP
    dst64 = _pad_edges(dst, ept64, E64, NS, DUMP)
    ept16 = N_EDGES // NW
    src16 = _pad_edges(src, ept16, E16, NW, 0)
    dst16 = _pad_edges(dst, ept16, E16, NW, DUMP)

    x_p = jnp.pad(x.astype(jnp.float32), ((0, NP - N_NODES), (0, 0)))
    x_stack = jnp.concatenate([x_p[:, :DH], x_p[:, DH:]], axis=0)
    zeros64 = jnp.zeros((K64, CHUNK, DH), jnp.float32)
    zeros16 = jnp.zeros((K16, CHUNK, 16), jnp.float32)
    zeros_np = jnp.zeros((NP,), jnp.float32)

    xr1 = _self(x_p, Wr1, b1.reshape(1, D))

    s1, deg = _seg64_deg(x_stack, src64, src64_hi, dst64, zeros64, zeros_np)
    h1 = _layer1(s1, deg, xr1, Wl1)

    hr2 = _self2(h1, Wr2, b2.reshape(1, D))
    s2 = _seg64(h1.reshape(2 * NP, DH), src64, src64_hi, dst64, zeros64)
    wl3_16 = jnp.pad(Wl3, ((0, 0), (0, 14)))
    wr3_16 = jnp.pad(Wr3, ((0, 0), (0, 14)))
    b3_16 = jnp.pad(b3, (0, 14)).reshape(1, 16)
    p16, q16 = _layer2(s2, deg, hr2, Wl2, wl3_16, wr3_16, b3_16)

    s3 = _seg16(p16, src16, src16, dst16, zeros16)
    out = _final(s3, deg, q16)
    return out[:N_NODES]


# speedup vs baseline: 1.0948x; 1.0094x over previous
import functools

import jax
import jax.numpy as jnp
from jax import lax
from jax.experimental import pallas as pl
from jax.experimental.pallas import tpu as pltpu
from jax.experimental.pallas import tpu_sc as plsc

N_NODES = 10000
N_EDGES = 320000
D = 128
DH = D // 2

NC = 2
NS = 16
NW = NC * NS
NP = 10016
RPT = NP // NS
DUMP = 10008
CHUNK = 128
L = 16

E64 = 20480
K64 = 2
E16 = 10240
K16 = 8

_SC_PARAMS = pltpu.CompilerParams(use_tc_tiling_on_sc=False,
                                  needs_layout_passes=False)


def _make_pass(width, ept, k, feature_split, with_deg):
    nch = ept // CHUNK
    nt = nch // (2 * k)
    mesh = plsc.VectorSubcoreMesh(core_axis_name="c", subcore_axis_name="s")

    out_type = jax.ShapeDtypeStruct((NC, NP, width), jnp.float32)
    scratch = [
        pltpu.VMEM((ept,), jnp.int32),
        pltpu.VMEM((ept,), jnp.int32),
        pltpu.VMEM((k, CHUNK, width), jnp.float32),
        pltpu.VMEM((k, CHUNK, width), jnp.float32),
        pltpu.VMEM_SHARED((NP, width), jnp.float32),
        pltpu.SemaphoreType.DMA,
        pltpu.SemaphoreType.DMA,
        pltpu.SemaphoreType.DMA,
        pltpu.SemaphoreType.DMA,
    ]
    if with_deg:
        out_type = [out_type, jax.ShapeDtypeStruct((NS, NP), jnp.float32)]
        scratch.insert(4, pltpu.VMEM((NP,), jnp.float32))

    @functools.partial(pl.kernel, out_type=out_type, mesh=mesh,
                       scratch_types=scratch, compiler_params=_SC_PARAMS)
    def seg(*refs):
        if with_deg:
            (table_hbm, srclo_hbm, srchi_hbm, dst_hbm, zeros_hbm, znp_hbm,
             out_hbm, deg_hbm,
             src_v, dst_v, buf0, buf1, deg_v, acc_sh,
             gsem0, gsem1, ssem0, ssem1) = refs
        else:
            (table_hbm, srclo_hbm, srchi_hbm, dst_hbm, zeros_hbm,
             out_hbm,
             src_v, dst_v, buf0, buf1, acc_sh,
             gsem0, gsem1, ssem0, ssem1) = refs
        c = lax.axis_index("c")
        s = lax.axis_index("s")
        if feature_split:
            base = s * ept

            @pl.when(c == 0)
            def _():
                pltpu.sync_copy(srclo_hbm.at[pl.ds(base, ept)], src_v)

            @pl.when(c == 1)
            def _():
                pltpu.sync_copy(srchi_hbm.at[pl.ds(base, ept)], src_v)
        else:
            base = (s * NC + c) * ept
            pltpu.sync_copy(srclo_hbm.at[pl.ds(base, ept)], src_v)
        pltpu.sync_copy(dst_hbm.at[pl.ds(base, ept)], dst_v)
        if with_deg:
            pltpu.sync_copy(znp_hbm, deg_v)

        def idx(ref, chunk_no):
            return ref.at[pl.ds(chunk_no * CHUNK, CHUNK)]

        def gather_k(group, buf, sem):
            for j in range(k):
                pltpu.async_copy(table_hbm.at[idx(src_v, group * k + j)],
                                 buf.at[j], sem)

        def gwait_k(buf, sem):
            d = pltpu.make_async_copy(table_hbm.at[idx(src_v, 0)],
                                      buf.at[0], sem)
            for _ in range(k):
                d.wait()

        def scatter_k(group, buf, sem):
            for j in range(k):
                pltpu.async_copy(buf.at[j],
                                 acc_sh.at[idx(dst_v, group * k + j)],
                                 sem, add=True)

        def swait_k(buf, sem):
            d = pltpu.make_async_copy(buf.at[0], acc_sh.at[idx(dst_v, 0)], sem)
            for _ in range(k):
                d.wait()

        gather_k(0, buf0, gsem0)
        pltpu.sync_copy(zeros_hbm, buf1)
        nfull = RPT // CHUNK
        for j in range(nfull):
            pltpu.sync_copy(buf1.at[j % k],
                            acc_sh.at[pl.ds(s * RPT + j * CHUNK, CHUNK)])
        rem = RPT - nfull * CHUNK
        if rem:
            pltpu.sync_copy(buf1.at[0].at[pl.ds(0, rem)],
                            acc_sh.at[pl.ds(s * RPT + nfull * CHUNK, rem)])
        plsc.subcore_barrier()

        epg = 2 * k * CHUNK

        def body(t, carry):
            g0 = 2 * t
            g1 = g0 + 1
            gather_k(g1, buf1, gsem1)
            gwait_k(buf0, gsem0)
            scatter_k(g0, buf0, ssem0)
            gwait_k(buf1, gsem1)
            scatter_k(g1, buf1, ssem1)
            if with_deg:
                @pl.when(c == 0)
                def _():
                    ones = jnp.full((L,), 1.0, jnp.float32)

                    def dbody(i, carry2):
                        dd = dst_v[pl.ds(t * epg + i * L, L)]
                        plsc.addupdate_scatter(deg_v, [dd], ones)
                        return carry2

                    lax.fori_loop(0, epg // L, dbody, 0)
            swait_k(buf0, ssem0)

            @pl.when(t + 1 < nt)
            def _():
                gather_k(g0 + 2, buf0, gsem0)

            swait_k(buf1, ssem1)
            return carry

        lax.fori_loop(0, nt, body, 0)
        plsc.subcore_barrier()
        pltpu.sync_copy(acc_sh.at[pl.ds(s * RPT, RPT)],
                        out_hbm.at[c, pl.ds(s * RPT, RPT)])
        if with_deg:
            @pl.when(c == 0)
            def _():
                pltpu.sync_copy(deg_v, deg_hbm.at[s])

    return seg


_seg64_deg = _make_pass(DH, E64, K64, True, True)
_seg64 = _make_pass(DH, E64, K64, True, False)
_seg16 = _make_pass(16, E16, K16, False, False)


def _inv_deg(deg_ref):
    deg = jnp.sum(deg_ref[...], axis=0)[:, None]
    return 1.0 / jnp.maximum(deg, 1.0)


def _layer1_body(acc_ref, deg_ref, x_ref, wl_ref, wr_ref, b_ref, o_ref):
    inv = _inv_deg(deg_ref)
    agg = jnp.concatenate([acc_ref[0], acc_ref[1]], axis=1) * inv
    h = (jnp.dot(agg, wl_ref[...], preferred_element_type=jnp.float32)
         + jnp.dot(x_ref[...], wr_ref[...], preferred_element_type=jnp.float32)
         + b_ref[...])
    h = jnp.maximum(h, 0.0)
    o_ref[0] = h[:, :DH]
    o_ref[1] = h[:, DH:]


def _layer1(acc, deg, x, wl, wr, b):
    return pl.pallas_call(
        _layer1_body,
        out_shape=jax.ShapeDtypeStruct((2, NP, DH), jnp.float32),
    )(acc, deg, x, wl, wr, b)


def _layer2_body(acc_ref, deg_ref, h_ref, wl_ref, wr_ref, b_ref,
                 wl3_ref, wr3_ref, b3_ref, p_ref, q_ref):
    inv = _inv_deg(deg_ref)
    agg = jnp.concatenate([acc_ref[0], acc_ref[1]], axis=1) * inv
    h1 = jnp.concatenate([h_ref[0], h_ref[1]], axis=1)
    h = (jnp.dot(agg, wl_ref[...], preferred_element_type=jnp.float32)
         + jnp.dot(h1, wr_ref[...], preferred_element_type=jnp.float32)
         + b_ref[...])
    h = jnp.maximum(h, 0.0)
    p_ref[...] = jnp.dot(h, wl3_ref[...], preferred_element_type=jnp.float32)
    q_ref[...] = (jnp.dot(h, wr3_ref[...], preferred_element_type=jnp.float32)
                  + b3_ref[...])


def _layer2(acc, deg, h, wl, wr, b, wl3, wr3, b3):
    return pl.pallas_call(
        _layer2_body,
        out_shape=[jax.ShapeDtypeStruct((NP, 16), jnp.float32),
                   jax.ShapeDtypeStruct((NP, 16), jnp.float32)],
    )(acc, deg, h, wl, wr, b, wl3, wr3, b3)


def _final_body(acc_ref, deg_ref, q_ref, o_ref):
    inv = _inv_deg(deg_ref)
    z = (acc_ref[0] + acc_ref[1]) * inv + q_ref[...]
    z0 = z[:, 0:1]
    z1 = z[:, 1:2]
    m = jnp.maximum(z0, z1)
    lse = m + jnp.log(jnp.exp(z0 - m) + jnp.exp(z1 - m))
    o_ref[...] = jnp.concatenate([z0 - lse, z1 - lse], axis=1)


def _final(acc, deg, q):
    return pl.pallas_call(
        _final_body,
        out_shape=jax.ShapeDtypeStruct((NP, 2), jnp.float32),
    )(acc, deg, q)


def _pad_edges(a, per_tile, per_tile_pad, ntiles, fill):
    a = a.reshape(ntiles, per_tile)
    return jnp.pad(a, ((0, 0), (0, per_tile_pad - per_tile)),
                   constant_values=fill).reshape(-1)


def kernel(x, edge_index, Wl1, Wr1, b1, Wl2, Wr2, b2, Wl3, Wr3, b3):
    ei = edge_index.astype(jnp.int32)
    src = ei[0]
    dst = ei[1]

    ept64 = N_EDGES // NS
    src64 = _pad_edges(src, ept64, E64, NS, 0)
    src64_hi = src64 + NP
    dst64 = _pad_edges(dst, ept64, E64, NS, DUMP)
    ept16 = N_EDGES // NW
    src16 = _pad_edges(src, ept16, E16, NW, 0)
    dst16 = _pad_edges(dst, ept16, E16, NW, DUMP)

    x_p = jnp.pad(x.astype(jnp.float32), ((0, NP - N_NODES), (0, 0)))
    x_stack = jnp.concatenate([x_p[:, :DH], x_p[:, DH:]], axis=0)
    zeros64 = jnp.zeros((K64, CHUNK, DH), jnp.float32)
    zeros16 = jnp.zeros((K16, CHUNK, 16), jnp.float32)
    zeros_np = jnp.zeros((NP,), jnp.float32)

    s1, deg = _seg64_deg(x_stack, src64, src64_hi, dst64, zeros64, zeros_np)
    h1 = _layer1(s1, deg, x_p, Wl1, Wr1, b1.reshape(1, D))

    s2 = _seg64(h1.reshape(2 * NP, DH), src64, src64_hi, dst64, zeros64)
    wl3_16 = jnp.pad(Wl3, ((0, 0), (0, 14)))
    wr3_16 = jnp.pad(Wr3, ((0, 0), (0, 14)))
    b3_16 = jnp.pad(b3, (0, 14)).reshape(1, 16)
    p16, q16 = _layer2(s2, deg, h1, Wl2, Wr2, b2.reshape(1, D),
                       wl3_16, wr3_16, b3_16)

    s3 = _seg16(p16, src16, src16, dst16, zeros16)
    out = _final(s3, deg, q16)
    return out[:N_NODES]
